# baseline (speedup 1.0000x reference)
import jax
import jax.numpy as jnp
from jax import lax
from jax.experimental import pallas as pl
from jax.experimental.pallas import tpu as pltpu

_USER = 4096
_N = 8192
_D = 16
_BM = 256
_NB = _N // _BM
_NBU = _USER // _BM
_DF = 2
_DB = 6


def _body(a_hbm, uf_ref, u1w_ref, usf_ref, u2w_ref, itf_ref, iw_ref,
          w1u_ref, w1i_ref, w2u_ref, w2i_ref, alpha_ref,
          user_ref, item_ref, abf_hbm,
          fbuf, bbuf, xs, in_sem, wsem, rsem):
    alpha = alpha_ref[0, 0]

    for b in range(_DF - 1):
        pltpu.make_async_copy(
            a_hbm.at[pl.ds(b * _BM, _BM), :], fbuf.at[b], in_sem.at[b]).start()

    ue1 = jnp.dot(uf_ref[...], u1w_ref[...], preferred_element_type=jnp.float32)
    ue2 = jnp.dot(usf_ref[...], u2w_ref[...], preferred_element_type=jnp.float32)
    ie = jnp.dot(itf_ref[...], iw_ref[...], preferred_element_type=jnp.float32)
    xs[0, :_USER, :] = jnp.concatenate([ue1, ue2], axis=1).astype(jnp.bfloat16)
    xs[0, _USER:, :] = ie.astype(jnp.bfloat16)

    def in_copy(i, b):
        return pltpu.make_async_copy(
            a_hbm.at[pl.ds(i * _BM, _BM), :], fbuf.at[b], in_sem.at[b])

    def out_copy(i, s):
        return pltpu.make_async_copy(
            bbuf.at[s], abf_hbm.at[pl.ds(i * _BM, _BM), :], wsem.at[s])

    def rd_copy(i, s):
        return pltpu.make_async_copy(
            abf_hbm.at[pl.ds(i * _BM, _BM), :], bbuf.at[s], rsem.at[s])

    def store_emb(i, p, emb):
        cols = slice(p * _D, (p + 1) * _D)

        @pl.when(i < _NBU)
        def _():
            user_ref[pl.ds(i * _BM, _BM), cols] = emb

        @pl.when(i >= _NBU)
        def _():
            item_ref[pl.ds((i - _NBU) * _BM, _BM), cols] = emb

    x0 = xs[0][...]
    w1u = w1u_ref[...]
    w1i = w1i_ref[...]

    def step1(i, carry):
        b = lax.rem(i, _DF)
        s = lax.rem(i, _DB)

        @pl.when(i + _DF - 1 < _NB)
        def _():
            in_copy(i + _DF - 1, lax.rem(i + _DF - 1, _DF)).start()

        @pl.when(i >= _DB)
        def _():
            out_copy(i - _DB, s).wait()

        in_copy(i, b).wait()
        a = fbuf[b][...].astype(jnp.bfloat16)
        bbuf[s] = a
        out_copy(i, s).start()
        acc = jnp.dot(a, x0, preferred_element_type=jnp.float32)
        emb = jnp.where(acc >= 0, acc, alpha * acc)
        store_emb(i, 0, emb)
        w = jnp.where(i < _NBU, w1u, w1i)
        xs[1, pl.ds(i * _BM, _BM), :] = jnp.dot(
            emb, w, preferred_element_type=jnp.float32).astype(jnp.bfloat16)
        return carry

    lax.fori_loop(0, _NB, step1, 0)
    for k in range(_DB):
        out_copy(_NB - _DB + k, k).wait()

    def stream_pass(p, wu, wi):
        for k in range(_DB - 1):
            rd_copy(k, k).start()
        x = xs[p][...]

        def step(i, carry):
            s = lax.rem(i, _DB)

            @pl.when(i + _DB - 1 < _NB)
            def _():
                rd_copy(i + _DB - 1, lax.rem(i + _DB - 1, _DB)).start()

            rd_copy(i, s).wait()
            acc = jnp.dot(bbuf[s][...], x, preferred_element_type=jnp.float32)
            emb = jnp.where(acc >= 0, acc, alpha * acc)
            store_emb(i, p, emb)
            if wu is not None:
                w = jnp.where(i < _NBU, wu, wi)
                xs[p + 1, pl.ds(i * _BM, _BM), :] = jnp.dot(
                    emb, w, preferred_element_type=jnp.float32
                ).astype(jnp.bfloat16)
            return carry

        lax.fori_loop(0, _NB, step, 0)

    stream_pass(1, w2u_ref[...], w2i_ref[...])
    stream_pass(2, None, None)


def kernel(user_social_feat, user_feat, item_feat, raitng_adj,
           user1_w, user2_w, item_w, user_w1, item_w1, user_w2, item_w2,
           prelu_a):
    alpha = jnp.reshape(prelu_a, (1, 1))
    user_emb, item_emb, _ = pl.pallas_call(
        _body,
        in_specs=[
            pl.BlockSpec(memory_space=pltpu.MemorySpace.HBM),
            pl.BlockSpec((_USER, 128), lambda: (0, 0)),
            pl.BlockSpec((128, _D // 2), lambda: (0, 0)),
            pl.BlockSpec((_USER, 128), lambda: (0, 0)),
            pl.BlockSpec((128, _D // 2), lambda: (0, 0)),
            pl.BlockSpec((_USER, 128), lambda: (0, 0)),
            pl.BlockSpec((128, _D), lambda: (0, 0)),
            pl.BlockSpec((_D, _D), lambda: (0, 0)),
            pl.BlockSpec((_D, _D), lambda: (0, 0)),
            pl.BlockSpec((_D, _D), lambda: (0, 0)),
            pl.BlockSpec((_D, _D), lambda: (0, 0)),
            pl.BlockSpec((1, 1), lambda: (0, 0)),
        ],
        out_specs=[
            pl.BlockSpec((_USER, 3 * _D), lambda: (0, 0)),
            pl.BlockSpec((_USER, 3 * _D), lambda: (0, 0)),
            pl.BlockSpec(memory_space=pltpu.MemorySpace.HBM),
        ],
        out_shape=[
            jax.ShapeDtypeStruct((_USER, 3 * _D), jnp.float32),
            jax.ShapeDtypeStruct((_USER, 3 * _D), jnp.float32),
            jax.ShapeDtypeStruct((_N, _N), jnp.bfloat16),
        ],
        scratch_shapes=[
            pltpu.VMEM((_DF, _BM, _N), jnp.float32),
            pltpu.VMEM((_DB, _BM, _N), jnp.bfloat16),
            pltpu.VMEM((3, _N, _D), jnp.bfloat16),
            pltpu.SemaphoreType.DMA((_DF,)),
            pltpu.SemaphoreType.DMA((_DB,)),
            pltpu.SemaphoreType.DMA((_DB,)),
        ],
        compiler_params=pltpu.CompilerParams(
            vmem_limit_bytes=100 * 1024 * 1024),
    )(raitng_adj, user_feat, user1_w, user_social_feat, user2_w, item_feat,
      item_w, user_w1, item_w1, user_w2, item_w2, alpha)
    return (user_emb, item_emb)


# speedup vs baseline: 1.0186x; 1.0186x over previous
import jax
import jax.numpy as jnp
from jax import lax
from jax.experimental import pallas as pl
from jax.experimental.pallas import tpu as pltpu

_USER = 4096
_N = 8192
_D = 16
_BM = 256
_NB = _N // _BM
_NBU = _USER // _BM
_DF = 2
_DB = 6
_K = 2
_NS = _NB - _K
_DR = _DB - _K


def _body(a_hbm, uf_ref, u1w_ref, usf_ref, u2w_ref, itf_ref, iw_ref,
          w1u_ref, w1i_ref, w2u_ref, w2i_ref, alpha_ref,
          user_ref, item_ref, abf_hbm,
          fbuf, bbuf, xs, in_sem, wsem, rsem):
    alpha = alpha_ref[0, 0]

    for b in range(_DF - 1):
        pltpu.make_async_copy(
            a_hbm.at[pl.ds(b * _BM, _BM), :], fbuf.at[b], in_sem.at[b]).start()

    ue1 = jnp.dot(uf_ref[...], u1w_ref[...], preferred_element_type=jnp.float32)
    ue2 = jnp.dot(usf_ref[...], u2w_ref[...], preferred_element_type=jnp.float32)
    ie = jnp.dot(itf_ref[...], iw_ref[...], preferred_element_type=jnp.float32)
    xs[0, :_USER, :] = jnp.concatenate([ue1, ue2], axis=1).astype(jnp.bfloat16)
    xs[0, _USER:, :] = ie.astype(jnp.bfloat16)

    def in_copy(i, b):
        return pltpu.make_async_copy(
            a_hbm.at[pl.ds(i * _BM, _BM), :], fbuf.at[b], in_sem.at[b])

    def out_copy(i, s):
        return pltpu.make_async_copy(
            bbuf.at[s], abf_hbm.at[pl.ds(i * _BM, _BM), :], wsem.at[s])

    def rd_copy(i, s):
        return pltpu.make_async_copy(
            abf_hbm.at[pl.ds(i * _BM, _BM), :], bbuf.at[s], rsem.at[s])

    def store_emb(i, p, emb):
        i = jnp.asarray(i, jnp.int32)
        cols = slice(p * _D, (p + 1) * _D)

        @pl.when(i < _NBU)
        def _():
            user_ref[pl.ds(i * _BM, _BM), cols] = emb

        @pl.when(i >= _NBU)
        def _():
            item_ref[pl.ds((i - _NBU) * _BM, _BM), cols] = emb

    x0 = xs[0][...]
    w1u = w1u_ref[...]
    w1i = w1i_ref[...]

    def step1(i, carry):
        b = lax.rem(i, _DF)
        s = lax.rem(i, _DB)

        @pl.when(i + _DF - 1 < _NB)
        def _():
            in_copy(i + _DF - 1, lax.rem(i + _DF - 1, _DF)).start()

        @pl.when(i >= _DB)
        def _():
            out_copy(i - _DB, s).wait()

        in_copy(i, b).wait()
        a = fbuf[b][...].astype(jnp.bfloat16)
        bbuf[s] = a

        @pl.when(i < _NS)
        def _():
            out_copy(i, s).start()

        acc = jnp.dot(a, x0, preferred_element_type=jnp.float32)
        emb = jnp.where(acc >= 0, acc, alpha * acc)
        store_emb(i, 0, emb)
        w = jnp.where(i < _NBU, w1u, w1i)
        xs[1, pl.ds(i * _BM, _BM), :] = jnp.dot(
            emb, w, preferred_element_type=jnp.float32).astype(jnp.bfloat16)
        return carry

    lax.fori_loop(0, _NB, step1, 0)
    for b in range(_NS - _DR, _NS):
        out_copy(b, b % _DB).wait()

    def compute_block(i, s, x, p, wu, wi):
        acc = jnp.dot(bbuf[s][...], x, preferred_element_type=jnp.float32)
        emb = jnp.where(acc >= 0, acc, alpha * acc)
        store_emb(i, p, emb)
        if wu is not None:
            w = jnp.where(i < _NBU, wu, wi)
            xs[p + 1, pl.ds(i * _BM, _BM), :] = jnp.dot(
                emb, w, preferred_element_type=jnp.float32
            ).astype(jnp.bfloat16)

    def stream_pass(p, wu, wi):
        for k in range(_DR - 1):
            rd_copy(k, _K + k).start()
        x = xs[p][...]

        def step(j, carry):
            s = _K + lax.rem(j, _DR)

            @pl.when(j + _DR - 1 < _NS)
            def _():
                rd_copy(j + _DR - 1, _K + lax.rem(j + _DR - 1, _DR)).start()

            rd_copy(j, s).wait()
            compute_block(j, s, x, p, wu, wi)
            return carry

        lax.fori_loop(0, _NS, step, 0, unroll=2)
        for t in range(_K):
            compute_block(_NS + t, t, x, p, wu, wi)

    stream_pass(1, w2u_ref[...], w2i_ref[...])
    stream_pass(2, None, None)


def kernel(user_social_feat, user_feat, item_feat, raitng_adj,
           user1_w, user2_w, item_w, user_w1, item_w1, user_w2, item_w2,
           prelu_a):
    alpha = jnp.reshape(prelu_a, (1, 1))
    user_emb, item_emb, _ = pl.pallas_call(
        _body,
        in_specs=[
            pl.BlockSpec(memory_space=pltpu.MemorySpace.HBM),
            pl.BlockSpec((_USER, 128), lambda: (0, 0)),
            pl.BlockSpec((128, _D // 2), lambda: (0, 0)),
            pl.BlockSpec((_USER, 128), lambda: (0, 0)),
            pl.BlockSpec((128, _D // 2), lambda: (0, 0)),
            pl.BlockSpec((_USER, 128), lambda: (0, 0)),
            pl.BlockSpec((128, _D), lambda: (0, 0)),
            pl.BlockSpec((_D, _D), lambda: (0, 0)),
            pl.BlockSpec((_D, _D), lambda: (0, 0)),
            pl.BlockSpec((_D, _D), lambda: (0, 0)),
            pl.BlockSpec((_D, _D), lambda: (0, 0)),
            pl.BlockSpec((1, 1), lambda: (0, 0)),
        ],
        out_specs=[
            pl.BlockSpec((_USER, 3 * _D), lambda: (0, 0)),
            pl.BlockSpec((_USER, 3 * _D), lambda: (0, 0)),
            pl.BlockSpec(memory_space=pltpu.MemorySpace.HBM),
        ],
        out_shape=[
            jax.ShapeDtypeStruct((_USER, 3 * _D), jnp.float32),
            jax.ShapeDtypeStruct((_USER, 3 * _D), jnp.float32),
            jax.ShapeDtypeStruct((_N, _N), jnp.bfloat16),
        ],
        scratch_shapes=[
            pltpu.VMEM((_DF, _BM, _N), jnp.float32),
            pltpu.VMEM((_DB, _BM, _N), jnp.bfloat16),
            pltpu.VMEM((3, _N, _D), jnp.bfloat16),
            pltpu.SemaphoreType.DMA((_DF,)),
            pltpu.SemaphoreType.DMA((_DB,)),
            pltpu.SemaphoreType.DMA((_DB,)),
        ],
        compiler_params=pltpu.CompilerParams(
            vmem_limit_bytes=100 * 1024 * 1024),
    )(raitng_adj, user_feat, user1_w, user_social_feat, user2_w, item_feat,
      item_w, user_w1, item_w1, user_w2, item_w2, alpha)
    return (user_emb, item_emb)


# speedup vs baseline: 1.0242x; 1.0054x over previous
import jax
import jax.numpy as jnp
from jax import lax
from jax.experimental import pallas as pl
from jax.experimental.pallas import tpu as pltpu

_USER = 4096
_N = 8192
_D = 16
_BM = 256
_NB = _N // _BM
_NBU = _USER // _BM
_DF = 2
_DB = 6
_K = 3
_NS = _NB - _K
_DR = _DB - _K
_RS0 = (_NS - _DR) % _DB


def _body(a_hbm, uf_ref, u1w_ref, usf_ref, u2w_ref, itf_ref, iw_ref,
          w1u_ref, w1i_ref, w2u_ref, w2i_ref, alpha_ref,
          user_ref, item_ref, abf_hbm,
          fbuf, bbuf, xs, in_sem, wsem, rsem):
    alpha = alpha_ref[0, 0]

    for b in range(_DF - 1):
        pltpu.make_async_copy(
            a_hbm.at[pl.ds(b * _BM, _BM), :], fbuf.at[b], in_sem.at[b]).start()

    ue1 = jnp.dot(uf_ref[...], u1w_ref[...], preferred_element_type=jnp.float32)
    ue2 = jnp.dot(usf_ref[...], u2w_ref[...], preferred_element_type=jnp.float32)
    ie = jnp.dot(itf_ref[...], iw_ref[...], preferred_element_type=jnp.float32)
    xs[0, :_USER, :] = jnp.concatenate([ue1, ue2], axis=1).astype(jnp.bfloat16)
    xs[0, _USER:, :] = ie.astype(jnp.bfloat16)

    def in_copy(i, b):
        return pltpu.make_async_copy(
            a_hbm.at[pl.ds(i * _BM, _BM), :], fbuf.at[b], in_sem.at[b])

    def out_copy(i, s):
        return pltpu.make_async_copy(
            bbuf.at[s], abf_hbm.at[pl.ds(i * _BM, _BM), :], wsem.at[s])

    def rd_copy(i, s):
        return pltpu.make_async_copy(
            abf_hbm.at[pl.ds(i * _BM, _BM), :], bbuf.at[s], rsem.at[s])

    def store_emb(i, p, emb):
        i = jnp.asarray(i, jnp.int32)
        cols = slice(p * _D, (p + 1) * _D)

        @pl.when(i < _NBU)
        def _():
            user_ref[pl.ds(i * _BM, _BM), cols] = emb

        @pl.when(i >= _NBU)
        def _():
            item_ref[pl.ds((i - _NBU) * _BM, _BM), cols] = emb

    x0 = xs[0][...]
    w1u = w1u_ref[...]
    w1i = w1i_ref[...]

    def step1(i, carry):
        b = lax.rem(i, _DF)
        s = lax.rem(i, _DB)

        @pl.when(i + _DF - 1 < _NB)
        def _():
            in_copy(i + _DF - 1, lax.rem(i + _DF - 1, _DF)).start()

        @pl.when(i >= _DB)
        def _():
            out_copy(i - _DB, s).wait()

        in_copy(i, b).wait()
        a = fbuf[b][...].astype(jnp.bfloat16)
        bbuf[s] = a

        @pl.when(i < _NS)
        def _():
            out_copy(i, s).start()

        acc = jnp.dot(a, x0, preferred_element_type=jnp.float32)
        emb = jnp.where(acc >= 0, acc, alpha * acc)
        store_emb(i, 0, emb)
        w = jnp.where(i < _NBU, w1u, w1i)
        xs[1, pl.ds(i * _BM, _BM), :] = jnp.dot(
            emb, w, preferred_element_type=jnp.float32).astype(jnp.bfloat16)
        return carry

    lax.fori_loop(0, _NB, step1, 0)
    for b in range(_NS - _DR, _NS):
        out_copy(b, b % _DB).wait()

    def compute_block(i, s, x, p, wu, wi):
        acc = jnp.dot(bbuf[s][...], x, preferred_element_type=jnp.float32)
        emb = jnp.where(acc >= 0, acc, alpha * acc)
        store_emb(i, p, emb)
        if wu is not None:
            w = jnp.where(i < _NBU, wu, wi)
            xs[p + 1, pl.ds(i * _BM, _BM), :] = jnp.dot(
                emb, w, preferred_element_type=jnp.float32
            ).astype(jnp.bfloat16)

    def stream_pass(p, wu, wi):
        for k in range(_DR - 1):
            rd_copy(k, _RS0 + k).start()
        x = xs[p][...]

        def step(j, carry):
            s = _RS0 + lax.rem(j, _DR)

            @pl.when(j + _DR - 1 < _NS)
            def _():
                rd_copy(j + _DR - 1, _RS0 + lax.rem(j + _DR - 1, _DR)).start()

            rd_copy(j, s).wait()
            compute_block(j, s, x, p, wu, wi)
            return carry

        lax.fori_loop(0, _NS, step, 0, unroll=2)
        for t in range(_K):
            compute_block(_NS + t, (_NS + t) % _DB, x, p, wu, wi)

    stream_pass(1, w2u_ref[...], w2i_ref[...])
    stream_pass(2, None, None)


def kernel(user_social_feat, user_feat, item_feat, raitng_adj,
           user1_w, user2_w, item_w, user_w1, item_w1, user_w2, item_w2,
           prelu_a):
    alpha = jnp.reshape(prelu_a, (1, 1))
    user_emb, item_emb, _ = pl.pallas_call(
        _body,
        in_specs=[
            pl.BlockSpec(memory_space=pltpu.MemorySpace.HBM),
            pl.BlockSpec((_USER, 128), lambda: (0, 0)),
            pl.BlockSpec((128, _D // 2), lambda: (0, 0)),
            pl.BlockSpec((_USER, 128), lambda: (0, 0)),
            pl.BlockSpec((128, _D // 2), lambda: (0, 0)),
            pl.BlockSpec((_USER, 128), lambda: (0, 0)),
            pl.BlockSpec((128, _D), lambda: (0, 0)),
            pl.BlockSpec((_D, _D), lambda: (0, 0)),
            pl.BlockSpec((_D, _D), lambda: (0, 0)),
            pl.BlockSpec((_D, _D), lambda: (0, 0)),
            pl.BlockSpec((_D, _D), lambda: (0, 0)),
            pl.BlockSpec((1, 1), lambda: (0, 0)),
        ],
        out_specs=[
            pl.BlockSpec((_USER, 3 * _D), lambda: (0, 0)),
            pl.BlockSpec((_USER, 3 * _D), lambda: (0, 0)),
            pl.BlockSpec(memory_space=pltpu.MemorySpace.HBM),
        ],
        out_shape=[
            jax.ShapeDtypeStruct((_USER, 3 * _D), jnp.float32),
            jax.ShapeDtypeStruct((_USER, 3 * _D), jnp.float32),
            jax.ShapeDtypeStruct((_N, _N), jnp.bfloat16),
        ],
        scratch_shapes=[
            pltpu.VMEM((_DF, _BM, _N), jnp.float32),
            pltpu.VMEM((_DB, _BM, _N), jnp.bfloat16),
            pltpu.VMEM((3, _N, _D), jnp.bfloat16),
            pltpu.SemaphoreType.DMA((_DF,)),
            pltpu.SemaphoreType.DMA((_DB,)),
            pltpu.SemaphoreType.DMA((_DB,)),
        ],
        compiler_params=pltpu.CompilerParams(
            vmem_limit_bytes=100 * 1024 * 1024),
    )(raitng_adj, user_feat, user1_w, user_social_feat, user2_w, item_feat,
      item_w, user_w1, item_w1, user_w2, item_w2, alpha)
    return (user_emb, item_emb)


# speedup vs baseline: 1.0274x; 1.0032x over previous
import jax
import jax.numpy as jnp
from jax import lax
from jax.experimental import pallas as pl
from jax.experimental.pallas import tpu as pltpu

_USER = 4096
_N = 8192
_D = 16
_BM = 256
_NB = _N // _BM
_NBU = _USER // _BM
_DF = 2
_DB = 6
_K = 3
_NS = _NB - _K
_DR = _DB - _K
_RS0 = (_NS - _DR) % _DB


def _body(a_hbm, uf_ref, u1w_ref, usf_ref, u2w_ref, itf_ref, iw_ref,
          w1u_ref, w1i_ref, w2u_ref, w2i_ref, alpha_ref,
          user_ref, item_ref, abf_hbm,
          fbuf, bbuf, xs, in_sem, wsem, rsem):
    alpha = alpha_ref[0, 0]

    for b in range(_DF - 1):
        pltpu.make_async_copy(
            a_hbm.at[pl.ds(b * _BM, _BM), :], fbuf.at[b], in_sem.at[b]).start()

    ue1 = jnp.dot(uf_ref[...], u1w_ref[...], preferred_element_type=jnp.float32)
    ue2 = jnp.dot(usf_ref[...], u2w_ref[...], preferred_element_type=jnp.float32)
    ie = jnp.dot(itf_ref[...], iw_ref[...], preferred_element_type=jnp.float32)
    xs[0, :_USER, :] = jnp.concatenate([ue1, ue2], axis=1).astype(jnp.bfloat16)
    xs[0, _USER:, :] = ie.astype(jnp.bfloat16)

    def in_copy(i, b):
        return pltpu.make_async_copy(
            a_hbm.at[pl.ds(i * _BM, _BM), :], fbuf.at[b], in_sem.at[b])

    def out_copy(i, s):
        return pltpu.make_async_copy(
            bbuf.at[s], abf_hbm.at[pl.ds(i * _BM, _BM), :], wsem.at[s])

    def rd_copy(i, s):
        return pltpu.make_async_copy(
            abf_hbm.at[pl.ds(i * _BM, _BM), :], bbuf.at[s], rsem.at[s])

    def store_emb(i, p, emb):
        i = jnp.asarray(i, jnp.int32)
        cols = slice(p * _D, (p + 1) * _D)

        @pl.when(i < _NBU)
        def _():
            user_ref[pl.ds(i * _BM, _BM), cols] = emb

        @pl.when(i >= _NBU)
        def _():
            item_ref[pl.ds((i - _NBU) * _BM, _BM), cols] = emb

    x0 = xs[0][...]
    w1u = w1u_ref[...]
    w1i = w1i_ref[...]

    def step1(i, carry):
        b = lax.rem(i, _DF)
        s = lax.rem(i, _DB)

        @pl.when(i + _DF - 1 < _NB)
        def _():
            in_copy(i + _DF - 1, lax.rem(i + _DF - 1, _DF)).start()

        @pl.when(i >= _DB)
        def _():
            out_copy(i - _DB, s).wait()

        in_copy(i, b).wait()
        a = fbuf[b][...].astype(jnp.bfloat16)
        bbuf[s] = a

        @pl.when(i < _NS)
        def _():
            out_copy(i, s).start()

        acc = jnp.dot(a, x0, preferred_element_type=jnp.float32)
        emb = jnp.where(acc >= 0, acc, alpha * acc)
        store_emb(i, 0, emb)
        w = jnp.where(i < _NBU, w1u, w1i)
        xs[1, pl.ds(i * _BM, _BM), :] = jnp.dot(
            emb, w, preferred_element_type=jnp.float32).astype(jnp.bfloat16)
        return carry

    lax.fori_loop(0, _NB, step1, 0)
    for b in range(_NS - _DR, _NS):
        out_copy(b, b % _DB).wait()

    def compute_block(i, s, x, p, wu, wi):
        acc = jnp.dot(bbuf[s][...], x, preferred_element_type=jnp.float32)
        emb = jnp.where(acc >= 0, acc, alpha * acc)
        store_emb(i, p, emb)
        if wu is not None:
            w = jnp.where(i < _NBU, wu, wi)
            xs[p + 1, pl.ds(i * _BM, _BM), :] = jnp.dot(
                emb, w, preferred_element_type=jnp.float32
            ).astype(jnp.bfloat16)

    def stream_pass(p, wu, wi):
        for k in range(_DR - 1):
            rd_copy(k, _RS0 + k).start()
        x = xs[p][...]

        def step(j, carry):
            s = _RS0 + lax.rem(j, _DR)

            @pl.when(j + _DR - 1 < _NS)
            def _():
                rd_copy(j + _DR - 1, _RS0 + lax.rem(j + _DR - 1, _DR)).start()

            rd_copy(j, s).wait()
            compute_block(j, s, x, p, wu, wi)
            return carry

        lax.fori_loop(0, _NS, step, 0, unroll=3)
        for t in range(_K):
            compute_block(_NS + t, (_NS + t) % _DB, x, p, wu, wi)

    stream_pass(1, w2u_ref[...], w2i_ref[...])
    stream_pass(2, None, None)


def kernel(user_social_feat, user_feat, item_feat, raitng_adj,
           user1_w, user2_w, item_w, user_w1, item_w1, user_w2, item_w2,
           prelu_a):
    alpha = jnp.reshape(prelu_a, (1, 1))
    user_emb, item_emb, _ = pl.pallas_call(
        _body,
        in_specs=[
            pl.BlockSpec(memory_space=pltpu.MemorySpace.HBM),
            pl.BlockSpec((_USER, 128), lambda: (0, 0)),
            pl.BlockSpec((128, _D // 2), lambda: (0, 0)),
            pl.BlockSpec((_USER, 128), lambda: (0, 0)),
            pl.BlockSpec((128, _D // 2), lambda: (0, 0)),
            pl.BlockSpec((_USER, 128), lambda: (0, 0)),
            pl.BlockSpec((128, _D), lambda: (0, 0)),
            pl.BlockSpec((_D, _D), lambda: (0, 0)),
            pl.BlockSpec((_D, _D), lambda: (0, 0)),
            pl.BlockSpec((_D, _D), lambda: (0, 0)),
            pl.BlockSpec((_D, _D), lambda: (0, 0)),
            pl.BlockSpec((1, 1), lambda: (0, 0)),
        ],
        out_specs=[
            pl.BlockSpec((_USER, 3 * _D), lambda: (0, 0)),
            pl.BlockSpec((_USER, 3 * _D), lambda: (0, 0)),
            pl.BlockSpec(memory_space=pltpu.MemorySpace.HBM),
        ],
        out_shape=[
            jax.ShapeDtypeStruct((_USER, 3 * _D), jnp.float32),
            jax.ShapeDtypeStruct((_USER, 3 * _D), jnp.float32),
            jax.ShapeDtypeStruct((_N, _N), jnp.bfloat16),
        ],
        scratch_shapes=[
            pltpu.VMEM((_DF, _BM, _N), jnp.float32),
            pltpu.VMEM((_DB, _BM, _N), jnp.bfloat16),
            pltpu.VMEM((3, _N, _D), jnp.bfloat16),
            pltpu.SemaphoreType.DMA((_DF,)),
            pltpu.SemaphoreType.DMA((_DB,)),
            pltpu.SemaphoreType.DMA((_DB,)),
        ],
        compiler_params=pltpu.CompilerParams(
            vmem_limit_bytes=100 * 1024 * 1024),
    )(raitng_adj, user_feat, user1_w, user_social_feat, user2_w, item_feat,
      item_w, user_w1, item_w1, user_w2, item_w2, alpha)
    return (user_emb, item_emb)
